# Initial kernel scaffold; baseline (speedup 1.0000x reference)
import jax
import jax.numpy as jnp
from jax.experimental import pallas as pl


def kernel(x, edge_index, edge_attr, trip_nodes, day, sec, nn0_W1, nn0_b1, nn0_W2, nn0_b2, lin0_W, conv0_b, nn1_W1, nn1_b1, nn1_W2, nn1_b2, lin1_W, conv1_b, mlp_W1, mlp_b1, mlp_W2, mlp_b2):
    raise NotImplementedError("write your pallas kernel here")


# speedup vs baseline: 3.6186x; 3.6186x over previous
import functools

import jax
import jax.numpy as jnp
from jax import lax
from jax.experimental import pallas as pl
from jax.experimental.pallas import tpu as pltpu
from jax.experimental.pallas import tpu_sc as plsc

_N = 10000
_E = 160000
_D = 16
_HID = 256

_NC = 2
_NS = 16
_NW = _NC * _NS
_CHUNK = 125
_CH = 40
_PER_W = _CH * _CHUNK
_NACC = _N
_ROWS_PER_TILE_Z = _NACC // _NS
_ROWS_PER_TILE_O = _N // _NS

_SC_PARAMS = dict(compiler_params=pltpu.CompilerParams(
    use_tc_tiling_on_sc=False))


@functools.lru_cache(maxsize=None)
def _get_mesh():
    return plsc.VectorSubcoreMesh(core_axis_name="c", subcore_axis_name="s",
                                  num_cores=_NC, num_subcores=_NS)


@functools.lru_cache(maxsize=None)
def _make_sc_gather():
    @functools.partial(
        pl.kernel,
        out_type=jax.ShapeDtypeStruct((_E, _D), jnp.float32),
        mesh=_get_mesh(),
        scratch_types=[
            pltpu.VMEM((_CH, _CHUNK), jnp.int32),
            pltpu.VMEM((_PER_W, _D), jnp.float32),
            pltpu.SemaphoreType.DMA,
        ],
        **_SC_PARAMS,
    )
    def k(h_hbm, src_hbm, out_hbm, idx_v, rows_v, sem):
        wid = lax.axis_index("s") * _NC + lax.axis_index("c")
        pltpu.sync_copy(src_hbm.at[wid], idx_v)

        def fire(j, _):
            pltpu.async_copy(
                h_hbm.at[idx_v.at[j]], rows_v.at[pl.ds(j * _CHUNK, _CHUNK)],
                sem)
            return 0

        lax.fori_loop(0, _CH, fire, 0)

        def drain(j, _):
            pltpu.make_async_copy(
                h_hbm.at[idx_v.at[j]], rows_v.at[pl.ds(j * _CHUNK, _CHUNK)],
                sem).wait()
            return 0

        lax.fori_loop(0, _CH, drain, 0)
        pltpu.sync_copy(rows_v, out_hbm.at[pl.ds(wid * _PER_W, _PER_W)])

    return k


def _sc_gather(h, src3):
    return _make_sc_gather()(h, src3)


@functools.lru_cache(maxsize=None)
def _make_sc_scatter():
    scratch = [
        pltpu.VMEM((_CH, _CHUNK), jnp.int32),
        pltpu.VMEM((_PER_W, _D), jnp.float32),
        pltpu.VMEM((_CHUNK, _D), jnp.float32),
        pltpu.VMEM_SHARED((_NACC, _D), jnp.float32),
    ]

    def body(msg_hbm, dst_hbm, sums_out, idx_v, msg_v, buf_v, acc):
        cid = lax.axis_index("c")
        sid = lax.axis_index("s")
        wid = sid * _NC + cid

        def zb(i, _):
            buf_v[i, :] = jnp.zeros((_D,), jnp.float32)
            return 0

        lax.fori_loop(0, _CHUNK, zb, 0)

        def zacc(i, _):
            base = sid * _ROWS_PER_TILE_Z + i * _CHUNK
            pltpu.sync_copy(buf_v, acc.at[pl.ds(base, _CHUNK)])
            return 0

        lax.fori_loop(0, _ROWS_PER_TILE_Z // _CHUNK, zacc, 0)
        rem = _ROWS_PER_TILE_Z - (_ROWS_PER_TILE_Z // _CHUNK) * _CHUNK
        if rem:
            pltpu.sync_copy(
                buf_v.at[pl.ds(0, rem)],
                acc.at[pl.ds(sid * _ROWS_PER_TILE_Z
                             + (_ROWS_PER_TILE_Z // _CHUNK) * _CHUNK, rem)])

        plsc.subcore_barrier()

        pltpu.sync_copy(msg_hbm.at[pl.ds(wid * _PER_W, _PER_W)], msg_v)
        pltpu.sync_copy(dst_hbm.at[wid], idx_v)

        def sc(j, _):
            pltpu.sync_copy(
                msg_v.at[pl.ds(j * _CHUNK, _CHUNK)], acc.at[idx_v.at[j]],
                add=True)
            return 0

        lax.fori_loop(0, _CH, sc, 0)
        plsc.subcore_barrier()

        base = sid * _ROWS_PER_TILE_O
        pltpu.sync_copy(acc.at[pl.ds(base, _ROWS_PER_TILE_O)],
                        sums_out.at[cid, pl.ds(base, _ROWS_PER_TILE_O)])

    return pl.kernel(
        body,
        out_type=jax.ShapeDtypeStruct((_NC, _N, _D), jnp.float32),
        mesh=_get_mesh(),
        scratch_types=scratch,
        **_SC_PARAMS,
    )


def _sc_scatter(msg, dst3):
    return _make_sc_scatter()(msg, dst3)


@functools.lru_cache(maxsize=None)
def _make_sc_count():
    scratch = [
        pltpu.VMEM((_CH, _CHUNK), jnp.int32),
        pltpu.VMEM((_CHUNK, _D), jnp.float32),
        pltpu.VMEM_SHARED((_NACC, _D), jnp.float32),
    ]

    def body(dst_hbm, cnt_out, idx_v, buf_v, acc):
        cid = lax.axis_index("c")
        sid = lax.axis_index("s")
        wid = sid * _NC + cid

        def zb(i, _):
            buf_v[i, :] = jnp.zeros((_D,), jnp.float32)
            return 0

        lax.fori_loop(0, _CHUNK, zb, 0)

        def zacc(i, _):
            base = sid * _ROWS_PER_TILE_Z + i * _CHUNK
            pltpu.sync_copy(buf_v, acc.at[pl.ds(base, _CHUNK)])
            return 0

        lax.fori_loop(0, _ROWS_PER_TILE_Z // _CHUNK, zacc, 0)
        rem = _ROWS_PER_TILE_Z - (_ROWS_PER_TILE_Z // _CHUNK) * _CHUNK
        if rem:
            pltpu.sync_copy(
                buf_v.at[pl.ds(0, rem)],
                acc.at[pl.ds(sid * _ROWS_PER_TILE_Z
                             + (_ROWS_PER_TILE_Z // _CHUNK) * _CHUNK, rem)])

        def ob(i, _):
            buf_v[i, :] = jnp.full((_D,), 1.0, jnp.float32)
            return 0

        lax.fori_loop(0, _CHUNK, ob, 0)
        plsc.subcore_barrier()

        pltpu.sync_copy(dst_hbm.at[wid], idx_v)

        def sc(j, _):
            pltpu.sync_copy(buf_v, acc.at[idx_v.at[j]], add=True)
            return 0

        lax.fori_loop(0, _CH, sc, 0)
        plsc.subcore_barrier()

        base = sid * _ROWS_PER_TILE_O
        pltpu.sync_copy(acc.at[pl.ds(base, _ROWS_PER_TILE_O)],
                        cnt_out.at[cid, pl.ds(base, _ROWS_PER_TILE_O)])

    return pl.kernel(
        body,
        out_type=jax.ShapeDtypeStruct((_NC, _N, _D), jnp.float32),
        mesh=_get_mesh(),
        scratch_types=scratch,
        **_SC_PARAMS,
    )


def _sc_count(dst3):
    return _make_sc_count()(dst3)


@functools.lru_cache(maxsize=None)
def _make_sc_trip_gather():
    @functools.partial(
        pl.kernel,
        out_type=jax.ShapeDtypeStruct((200, _D), jnp.float32),
        mesh=_get_mesh(),
        scratch_types=[
            pltpu.VMEM((2, 128), jnp.int32),
            pltpu.VMEM((200, _D), jnp.float32),
            pltpu.SemaphoreType.DMA,
        ],
        **_SC_PARAMS,
    )
    def k(h_hbm, trip_hbm, out_hbm, idx_v, rows_v, sem):
        wid = lax.axis_index("s") * _NC + lax.axis_index("c")

        @pl.when(wid == 0)
        def _():
            pltpu.sync_copy(trip_hbm.at[pl.ds(0, 128)], idx_v.at[0])
            pltpu.sync_copy(trip_hbm.at[pl.ds(128, 72)],
                            idx_v.at[1, pl.ds(0, 72)])
            pltpu.async_copy(h_hbm.at[idx_v.at[0]],
                             rows_v.at[pl.ds(0, 128)], sem).wait()
            pltpu.async_copy(h_hbm.at[idx_v.at[1, pl.ds(0, 72)]],
                             rows_v.at[pl.ds(128, 72)], sem).wait()
            pltpu.sync_copy(rows_v, out_hbm)

    return k


def _sc_trip_gather(h, trip):
    return _make_sc_trip_gather()(h, trip)


_BM = 2000


def _tc_msg_body(ea_ref, xs_ref, w1_ref, b1_ref, w2_ref, b2_ref, r_ref, s_ref,
                 out_ref):
    ea = ea_ref[...]
    hmid = jnp.maximum(
        jnp.dot(ea, w1_ref[...], preferred_element_type=jnp.float32)
        + b1_ref[...], 0.0)
    w = jnp.dot(hmid, w2_ref[...], preferred_element_type=jnp.float32) + b2_ref[...]
    xsrep = jnp.dot(xs_ref[...], r_ref[...], preferred_element_type=jnp.float32)
    out_ref[...] = jnp.dot(xsrep * w, s_ref[...],
                           preferred_element_type=jnp.float32)


def _tc_msg(ea, xs, w1, b1, w2, b2, rmat, smat):
    grid = _E // _BM
    return pl.pallas_call(
        _tc_msg_body,
        grid=(grid,),
        in_specs=[
            pl.BlockSpec((_BM, _D), lambda i: (i, 0)),
            pl.BlockSpec((_BM, _D), lambda i: (i, 0)),
            pl.BlockSpec((_D, _HID), lambda i: (0, 0)),
            pl.BlockSpec((1, _HID), lambda i: (0, 0)),
            pl.BlockSpec((_HID, _HID), lambda i: (0, 0)),
            pl.BlockSpec((1, _HID), lambda i: (0, 0)),
            pl.BlockSpec((_D, _HID), lambda i: (0, 0)),
            pl.BlockSpec((_HID, _D), lambda i: (0, 0)),
        ],
        out_specs=pl.BlockSpec((_BM, _D), lambda i: (i, 0)),
        out_shape=jax.ShapeDtypeStruct((_E, _D), jnp.float32),
        compiler_params=pltpu.CompilerParams(
            dimension_semantics=("parallel",)),
    )(ea, xs, w1, b1, w2, b2, rmat, smat)


def _tc_update_body(relu, sums_ref, cnt_ref, h_ref, wr_ref, b_ref, out_ref):
    s = sums_ref[0] + sums_ref[1]
    c = cnt_ref[0] + cnt_ref[1]
    cc = jnp.maximum(c[:, 0:1], 1.0)
    res = s / cc + jnp.dot(h_ref[...], wr_ref[...],
                           preferred_element_type=jnp.float32) + b_ref[...]
    if relu:
        res = jnp.maximum(res, 0.0)
    out_ref[...] = res


def _tc_update(sums_p, cnt_p, h, wr, b, relu):
    return pl.pallas_call(
        functools.partial(_tc_update_body, relu),
        out_shape=jax.ShapeDtypeStruct((_N, _D), jnp.float32),
    )(sums_p, cnt_p, h, wr, b)


def _tc_head_body(rows_ref, day_ref, sec_ref, w1a_ref, w1b_ref, w1c_ref,
                  b1_ref, w2_ref, b2_ref, out_ref):
    pooled = jnp.mean(rows_ref[...], axis=0, keepdims=True)
    hm = (jnp.dot(pooled, w1a_ref[...], preferred_element_type=jnp.float32)
          + day_ref[...] * w1b_ref[...]
          + sec_ref[...] * w1c_ref[...]
          + b1_ref[...])
    hm = jnp.maximum(hm, 0.0)
    out_ref[...] = jnp.dot(hm, w2_ref[...],
                           preferred_element_type=jnp.float32) + b2_ref[...]


def _tc_head(rows, day, sec, w1a, w1b, w1c, b1, w2, b2):
    return pl.pallas_call(
        _tc_head_body,
        out_shape=jax.ShapeDtypeStruct((1, 1), jnp.float32),
    )(rows, day, sec, w1a, w1b, w1c, b1, w2, b2)


def kernel(x, edge_index, edge_attr, trip_nodes, day, sec,
           nn0_W1, nn0_b1, nn0_W2, nn0_b2, lin0_W, conv0_b,
           nn1_W1, nn1_b1, nn1_W2, nn1_b2, lin1_W, conv1_b,
           mlp_W1, mlp_b1, mlp_W2, mlp_b2):
    src3 = edge_index[0].reshape(_NW, _CH, _CHUNK)
    dst3 = edge_index[1].reshape(_NW, _CH, _CHUNK)

    ii = lax.broadcasted_iota(jnp.int32, (_D, _HID), 0)
    kk = lax.broadcasted_iota(jnp.int32, (_D, _HID), 1)
    rmat = (kk // _D == ii).astype(jnp.float32)
    k2 = lax.broadcasted_iota(jnp.int32, (_HID, _D), 0)
    oo = lax.broadcasted_iota(jnp.int32, (_HID, _D), 1)
    smat = (k2 % _D == oo).astype(jnp.float32)

    xs0 = _sc_gather(x, src3)
    cnt = _sc_count(dst3)
    msg0 = _tc_msg(edge_attr, xs0, nn0_W1, nn0_b1.reshape(1, _HID),
                   nn0_W2, nn0_b2.reshape(1, _HID), rmat, smat)
    sums0 = _sc_scatter(msg0, dst3)
    h1 = _tc_update(sums0, cnt, x, lin0_W, conv0_b.reshape(1, _D), True)

    xs1 = _sc_gather(h1, src3)
    msg1 = _tc_msg(edge_attr, xs1, nn1_W1, nn1_b1.reshape(1, _HID),
                   nn1_W2, nn1_b2.reshape(1, _HID), rmat, smat)
    sums1 = _sc_scatter(msg1, dst3)
    h2 = _tc_update(sums1, cnt, h1, lin1_W, conv1_b.reshape(1, _D), False)

    rows = _sc_trip_gather(h2, trip_nodes)
    out = _tc_head(rows, day.reshape(1, 1), sec.reshape(1, 1),
                   mlp_W1[0:_D], mlp_W1[_D:_D + 1], mlp_W1[_D + 1:_D + 2],
                   mlp_b1.reshape(1, 64), mlp_W2, mlp_b2.reshape(1, 1))
    return out.reshape(1)


# speedup vs baseline: 3.7193x; 1.0278x over previous
import functools

import jax
import jax.numpy as jnp
from jax import lax
from jax.experimental import pallas as pl
from jax.experimental.pallas import tpu as pltpu
from jax.experimental.pallas import tpu_sc as plsc

_N = 10000
_E = 160000
_D = 16
_HID = 256

_NC = 2
_NS = 16
_NW = _NC * _NS
_CHUNK = 125
_CH = 40
_PER_W = _CH * _CHUNK
_NACC = _N
_ROWS_PER_TILE_Z = _NACC // _NS
_ROWS_PER_TILE_O = _N // _NS

_SC_PARAMS = dict(compiler_params=pltpu.CompilerParams(
    use_tc_tiling_on_sc=False))


@functools.lru_cache(maxsize=None)
def _get_mesh():
    return plsc.VectorSubcoreMesh(core_axis_name="c", subcore_axis_name="s",
                                  num_cores=_NC, num_subcores=_NS)


@functools.lru_cache(maxsize=None)
def _make_sc_gather():
    @functools.partial(
        pl.kernel,
        out_type=jax.ShapeDtypeStruct((_E, _D), jnp.float32),
        mesh=_get_mesh(),
        scratch_types=[
            pltpu.VMEM((_CH, _CHUNK), jnp.int32),
            pltpu.VMEM((_PER_W, _D), jnp.float32),
            pltpu.SemaphoreType.DMA,
        ],
        **_SC_PARAMS,
    )
    def k(h_hbm, src_hbm, out_hbm, idx_v, rows_v, sem):
        wid = lax.axis_index("s") * _NC + lax.axis_index("c")
        pltpu.sync_copy(src_hbm.at[wid], idx_v)

        def fire(j, _):
            pltpu.async_copy(
                h_hbm.at[idx_v.at[j]], rows_v.at[pl.ds(j * _CHUNK, _CHUNK)],
                sem)
            return 0

        lax.fori_loop(0, _CH, fire, 0)

        def drain(j, _):
            pltpu.make_async_copy(
                h_hbm.at[idx_v.at[j]], rows_v.at[pl.ds(j * _CHUNK, _CHUNK)],
                sem).wait()
            return 0

        lax.fori_loop(0, _CH, drain, 0)
        pltpu.sync_copy(rows_v, out_hbm.at[pl.ds(wid * _PER_W, _PER_W)])

    return k


def _sc_gather(h, src3):
    return _make_sc_gather()(h, src3)


@functools.lru_cache(maxsize=None)
def _make_sc_scatter():
    scratch = [
        pltpu.VMEM((_CH, _CHUNK), jnp.int32),
        pltpu.VMEM((_PER_W, _D), jnp.float32),
        pltpu.VMEM((_CHUNK, _D), jnp.float32),
        pltpu.VMEM_SHARED((_NACC, _D), jnp.float32),
    ]

    def body(msg_hbm, dst_hbm, sums_out, idx_v, msg_v, buf_v, acc):
        cid = lax.axis_index("c")
        sid = lax.axis_index("s")
        wid = sid * _NC + cid

        def zb(i, _):
            buf_v[i, :] = jnp.zeros((_D,), jnp.float32)
            return 0

        lax.fori_loop(0, _CHUNK, zb, 0)

        def zacc(i, _):
            base = sid * _ROWS_PER_TILE_Z + i * _CHUNK
            pltpu.sync_copy(buf_v, acc.at[pl.ds(base, _CHUNK)])
            return 0

        lax.fori_loop(0, _ROWS_PER_TILE_Z // _CHUNK, zacc, 0)
        rem = _ROWS_PER_TILE_Z - (_ROWS_PER_TILE_Z // _CHUNK) * _CHUNK
        if rem:
            pltpu.sync_copy(
                buf_v.at[pl.ds(0, rem)],
                acc.at[pl.ds(sid * _ROWS_PER_TILE_Z
                             + (_ROWS_PER_TILE_Z // _CHUNK) * _CHUNK, rem)])

        plsc.subcore_barrier()

        pltpu.sync_copy(msg_hbm.at[pl.ds(wid * _PER_W, _PER_W)], msg_v)
        pltpu.sync_copy(dst_hbm.at[wid], idx_v)

        def sc(j, _):
            pltpu.sync_copy(
                msg_v.at[pl.ds(j * _CHUNK, _CHUNK)], acc.at[idx_v.at[j]],
                add=True)
            return 0

        lax.fori_loop(0, _CH, sc, 0)
        plsc.subcore_barrier()

        base = sid * _ROWS_PER_TILE_O
        pltpu.sync_copy(acc.at[pl.ds(base, _ROWS_PER_TILE_O)],
                        sums_out.at[cid, pl.ds(base, _ROWS_PER_TILE_O)])

    return pl.kernel(
        body,
        out_type=jax.ShapeDtypeStruct((_NC, _N, _D), jnp.float32),
        mesh=_get_mesh(),
        scratch_types=scratch,
        **_SC_PARAMS,
    )


def _sc_scatter(msg, dst3):
    return _make_sc_scatter()(msg, dst3)


@functools.lru_cache(maxsize=None)
def _make_sc_count():
    scratch = [
        pltpu.VMEM((_CH, _CHUNK), jnp.int32),
        pltpu.VMEM((_CHUNK, _D), jnp.float32),
        pltpu.VMEM_SHARED((_NACC, _D), jnp.float32),
    ]

    def body(dst_hbm, cnt_out, idx_v, buf_v, acc):
        cid = lax.axis_index("c")
        sid = lax.axis_index("s")
        wid = sid * _NC + cid

        def zb(i, _):
            buf_v[i, :] = jnp.zeros((_D,), jnp.float32)
            return 0

        lax.fori_loop(0, _CHUNK, zb, 0)

        def zacc(i, _):
            base = sid * _ROWS_PER_TILE_Z + i * _CHUNK
            pltpu.sync_copy(buf_v, acc.at[pl.ds(base, _CHUNK)])
            return 0

        lax.fori_loop(0, _ROWS_PER_TILE_Z // _CHUNK, zacc, 0)
        rem = _ROWS_PER_TILE_Z - (_ROWS_PER_TILE_Z // _CHUNK) * _CHUNK
        if rem:
            pltpu.sync_copy(
                buf_v.at[pl.ds(0, rem)],
                acc.at[pl.ds(sid * _ROWS_PER_TILE_Z
                             + (_ROWS_PER_TILE_Z // _CHUNK) * _CHUNK, rem)])

        def ob(i, _):
            buf_v[i, :] = jnp.full((_D,), 1.0, jnp.float32)
            return 0

        lax.fori_loop(0, _CHUNK, ob, 0)
        plsc.subcore_barrier()

        pltpu.sync_copy(dst_hbm.at[wid], idx_v)

        def sc(j, _):
            pltpu.sync_copy(buf_v, acc.at[idx_v.at[j]], add=True)
            return 0

        lax.fori_loop(0, _CH, sc, 0)
        plsc.subcore_barrier()

        base = sid * _ROWS_PER_TILE_O
        pltpu.sync_copy(acc.at[pl.ds(base, _ROWS_PER_TILE_O)],
                        cnt_out.at[cid, pl.ds(base, _ROWS_PER_TILE_O)])

    return pl.kernel(
        body,
        out_type=jax.ShapeDtypeStruct((_NC, _N, _D), jnp.float32),
        mesh=_get_mesh(),
        scratch_types=scratch,
        **_SC_PARAMS,
    )


def _sc_count(dst3):
    return _make_sc_count()(dst3)


@functools.lru_cache(maxsize=None)
def _make_sc_trip_gather():
    out_type = [
        jax.ShapeDtypeStruct((2 * 200, _D), jnp.float32),
        jax.ShapeDtypeStruct((2 * 200, _D), jnp.float32),
        jax.ShapeDtypeStruct((200, _D), jnp.float32),
    ]

    @functools.partial(
        pl.kernel,
        out_type=out_type,
        mesh=_get_mesh(),
        scratch_types=[
            pltpu.VMEM((4, 128), jnp.int32),
            pltpu.VMEM((2 * 200, _D), jnp.float32),
            pltpu.VMEM((2 * 200, _D), jnp.float32),
            pltpu.VMEM((200, _D), jnp.float32),
            pltpu.SemaphoreType.DMA,
        ],
        **_SC_PARAMS,
    )
    def k(sums_hbm, cnt_hbm, h_hbm, trip_hbm, s_out, c_out, h_out,
          idx_v, s_v, c_v, h_v, sem):
        wid = lax.axis_index("s") * _NC + lax.axis_index("c")

        @pl.when(wid == 0)
        def _():
            pltpu.sync_copy(trip_hbm.at[0, pl.ds(0, 128)], idx_v.at[0])
            pltpu.sync_copy(trip_hbm.at[0, pl.ds(128, 72)],
                            idx_v.at[1, pl.ds(0, 72)])
            pltpu.sync_copy(trip_hbm.at[1, pl.ds(0, 128)], idx_v.at[2])
            pltpu.sync_copy(trip_hbm.at[1, pl.ds(128, 72)],
                            idx_v.at[3, pl.ds(0, 72)])
            for ref, dst in ((sums_hbm, s_v), (cnt_hbm, c_v)):
                pltpu.async_copy(ref.at[idx_v.at[0]],
                                 dst.at[pl.ds(0, 128)], sem)
                pltpu.async_copy(ref.at[idx_v.at[1, pl.ds(0, 72)]],
                                 dst.at[pl.ds(128, 72)], sem)
                pltpu.async_copy(ref.at[idx_v.at[2]],
                                 dst.at[pl.ds(200, 128)], sem)
                pltpu.async_copy(ref.at[idx_v.at[3, pl.ds(0, 72)]],
                                 dst.at[pl.ds(328, 72)], sem)
            pltpu.async_copy(h_hbm.at[idx_v.at[0]],
                             h_v.at[pl.ds(0, 128)], sem)
            pltpu.async_copy(h_hbm.at[idx_v.at[1, pl.ds(0, 72)]],
                             h_v.at[pl.ds(128, 72)], sem)
            pltpu.make_async_copy(h_hbm.at[idx_v.at[1, pl.ds(0, 72)]],
                                  h_v.at[pl.ds(128, 72)], sem).wait()
            pltpu.make_async_copy(h_hbm.at[idx_v.at[0]],
                                  h_v.at[pl.ds(0, 128)], sem).wait()
            for ref, dst in ((sums_hbm, s_v), (cnt_hbm, c_v)):
                pltpu.make_async_copy(ref.at[idx_v.at[0]],
                                      dst.at[pl.ds(0, 128)], sem).wait()
                pltpu.make_async_copy(ref.at[idx_v.at[1, pl.ds(0, 72)]],
                                      dst.at[pl.ds(128, 72)], sem).wait()
                pltpu.make_async_copy(ref.at[idx_v.at[2]],
                                      dst.at[pl.ds(200, 128)], sem).wait()
                pltpu.make_async_copy(ref.at[idx_v.at[3, pl.ds(0, 72)]],
                                      dst.at[pl.ds(328, 72)], sem).wait()
            pltpu.sync_copy(s_v, s_out)
            pltpu.sync_copy(c_v, c_out)
            pltpu.sync_copy(h_v, h_out)

    return k


def _sc_trip_gather(sums2n, cnt2n, h1, trip2):
    return _make_sc_trip_gather()(sums2n, cnt2n, h1, trip2)


_BM = 2000


def _tc_msg_body(ea_ref, xs_ref, w1_ref, b1_ref, w2_ref, b2_ref, r_ref, s_ref,
                 out_ref):
    ea = ea_ref[...]
    hmid = jnp.maximum(
        jnp.dot(ea, w1_ref[...], preferred_element_type=jnp.float32)
        + b1_ref[...], 0.0)
    w = jnp.dot(hmid, w2_ref[...], preferred_element_type=jnp.float32) + b2_ref[...]
    xsrep = jnp.dot(xs_ref[...], r_ref[...], preferred_element_type=jnp.float32)
    out_ref[...] = jnp.dot(xsrep * w, s_ref[...],
                           preferred_element_type=jnp.float32)


def _tc_msg(ea, xs, w1, b1, w2, b2, rmat, smat):
    grid = _E // _BM
    return pl.pallas_call(
        _tc_msg_body,
        grid=(grid,),
        in_specs=[
            pl.BlockSpec((_BM, _D), lambda i: (i, 0)),
            pl.BlockSpec((_BM, _D), lambda i: (i, 0)),
            pl.BlockSpec((_D, _HID), lambda i: (0, 0)),
            pl.BlockSpec((1, _HID), lambda i: (0, 0)),
            pl.BlockSpec((_HID, _HID), lambda i: (0, 0)),
            pl.BlockSpec((1, _HID), lambda i: (0, 0)),
            pl.BlockSpec((_D, _HID), lambda i: (0, 0)),
            pl.BlockSpec((_HID, _D), lambda i: (0, 0)),
        ],
        out_specs=pl.BlockSpec((_BM, _D), lambda i: (i, 0)),
        out_shape=jax.ShapeDtypeStruct((_E, _D), jnp.float32),
        compiler_params=pltpu.CompilerParams(
            dimension_semantics=("parallel",)),
    )(ea, xs, w1, b1, w2, b2, rmat, smat)


def _tc_update_body(relu, sums_ref, cnt_ref, h_ref, wr_ref, b_ref, out_ref):
    s = sums_ref[0] + sums_ref[1]
    c = cnt_ref[0] + cnt_ref[1]
    cc = jnp.maximum(c[:, 0:1], 1.0)
    res = s / cc + jnp.dot(h_ref[...], wr_ref[...],
                           preferred_element_type=jnp.float32) + b_ref[...]
    if relu:
        res = jnp.maximum(res, 0.0)
    out_ref[...] = res


def _tc_update(sums_p, cnt_p, h, wr, b, relu):
    return pl.pallas_call(
        functools.partial(_tc_update_body, relu),
        out_shape=jax.ShapeDtypeStruct((_N, _D), jnp.float32),
    )(sums_p, cnt_p, h, wr, b)


def _tc_head_body(s_ref, c_ref, h_ref, wr_ref, br_ref, day_ref, sec_ref,
                  w1a_ref, w1b_ref, w1c_ref, b1_ref, w2_ref, b2_ref, out_ref):
    s = s_ref[0:200] + s_ref[200:400]
    c = c_ref[0:200] + c_ref[200:400]
    cc = jnp.maximum(c[:, 0:1], 1.0)
    h2 = s / cc + jnp.dot(h_ref[...], wr_ref[...],
                          preferred_element_type=jnp.float32) + br_ref[...]
    pooled = jnp.mean(h2, axis=0, keepdims=True)
    hm = (jnp.dot(pooled, w1a_ref[...], preferred_element_type=jnp.float32)
          + day_ref[...] * w1b_ref[...]
          + sec_ref[...] * w1c_ref[...]
          + b1_ref[...])
    hm = jnp.maximum(hm, 0.0)
    out_ref[...] = jnp.dot(hm, w2_ref[...],
                           preferred_element_type=jnp.float32) + b2_ref[...]


def _tc_head(strip, ctrip, htrip, wr, br, day, sec, w1a, w1b, w1c, b1, w2, b2):
    return pl.pallas_call(
        _tc_head_body,
        out_shape=jax.ShapeDtypeStruct((1, 1), jnp.float32),
    )(strip, ctrip, htrip, wr, br, day, sec, w1a, w1b, w1c, b1, w2, b2)


def kernel(x, edge_index, edge_attr, trip_nodes, day, sec,
           nn0_W1, nn0_b1, nn0_W2, nn0_b2, lin0_W, conv0_b,
           nn1_W1, nn1_b1, nn1_W2, nn1_b2, lin1_W, conv1_b,
           mlp_W1, mlp_b1, mlp_W2, mlp_b2):
    src3 = edge_index[0].reshape(_NW, _CH, _CHUNK)
    dst3 = edge_index[1].reshape(_NW, _CH, _CHUNK)

    ii = lax.broadcasted_iota(jnp.int32, (_D, _HID), 0)
    kk = lax.broadcasted_iota(jnp.int32, (_D, _HID), 1)
    rmat = (kk // _D == ii).astype(jnp.float32)
    k2 = lax.broadcasted_iota(jnp.int32, (_HID, _D), 0)
    oo = lax.broadcasted_iota(jnp.int32, (_HID, _D), 1)
    smat = (k2 % _D == oo).astype(jnp.float32)

    xs0 = _sc_gather(x, src3)
    cnt = _sc_count(dst3)
    msg0 = _tc_msg(edge_attr, xs0, nn0_W1, nn0_b1.reshape(1, _HID),
                   nn0_W2, nn0_b2.reshape(1, _HID), rmat, smat)
    sums0 = _sc_scatter(msg0, dst3)
    h1 = _tc_update(sums0, cnt, x, lin0_W, conv0_b.reshape(1, _D), True)

    xs1 = _sc_gather(h1, src3)
    msg1 = _tc_msg(edge_attr, xs1, nn1_W1, nn1_b1.reshape(1, _HID),
                   nn1_W2, nn1_b2.reshape(1, _HID), rmat, smat)
    sums1 = _sc_scatter(msg1, dst3)

    trip2 = jnp.stack([trip_nodes, trip_nodes + _N])
    strip, ctrip, htrip = _sc_trip_gather(
        sums1.reshape(2 * _N, _D), cnt.reshape(2 * _N, _D), h1, trip2)
    out = _tc_head(strip, ctrip, htrip, lin1_W, conv1_b.reshape(1, _D),
                   day.reshape(1, 1), sec.reshape(1, 1),
                   mlp_W1[0:_D], mlp_W1[_D:_D + 1], mlp_W1[_D + 1:_D + 2],
                   mlp_b1.reshape(1, 64), mlp_W2, mlp_b2.reshape(1, 1))
    return out.reshape(1)
